# Initial kernel scaffold; baseline (speedup 1.0000x reference)
import jax
import jax.numpy as jnp
from jax.experimental import pallas as pl


def kernel(t, gamma):
    raise NotImplementedError("write your pallas kernel here")


# speedup vs baseline: 4.4611x; 4.4611x over previous
import functools

import jax
import jax.numpy as jnp
from jax import lax
from jax.experimental import pallas as pl
from jax.experimental.pallas import tpu as pltpu
from jax.experimental.pallas import tpu_sc as plsc

_TIMESTEPS = 1000
_B = 16384
_NC = 2
_NS = 16
_LANES = 16
_NW = _NC * _NS
_CHUNK = _B // _NW
_NVEC = _CHUNK // _LANES
_TAB_PAD = 1024
_MAGIC = 2.0 ** 23


@functools.partial(
    pl.kernel,
    mesh=plsc.VectorSubcoreMesh(core_axis_name="c", subcore_axis_name="s"),
    out_type=jax.ShapeDtypeStruct((_B,), jnp.float32),
    compiler_params=pltpu.CompilerParams(needs_layout_passes=False),
    scratch_types=[
        pltpu.VMEM((_TAB_PAD,), jnp.float32),
        pltpu.VMEM((_CHUNK,), jnp.float32),
        pltpu.VMEM((_CHUNK,), jnp.float32),
    ],
)
def _lookup(t_hbm, gamma_hbm, out_hbm, gamma_v, t_v, out_v):
    wid = lax.axis_index("s") * _NC + lax.axis_index("c")
    base = wid * _CHUNK
    pltpu.sync_copy(gamma_hbm, gamma_v)
    pltpu.sync_copy(t_hbm.at[pl.ds(base, _CHUNK)], t_v)
    for j in range(_NVEC):
        tv = t_v[pl.ds(j * _LANES, _LANES)]
        x = tv * jnp.float32(_TIMESTEPS)
        r = (x + jnp.float32(_MAGIC)) - jnp.float32(_MAGIC)
        idx = r.astype(jnp.int32)
        idx = jnp.minimum(jnp.maximum(idx, 0), _TIMESTEPS)
        out_v[pl.ds(j * _LANES, _LANES)] = plsc.load_gather(gamma_v, [idx])
    pltpu.sync_copy(out_v, out_hbm.at[pl.ds(base, _CHUNK)])


def kernel(t, gamma):
    gamma_p = jnp.zeros((_TAB_PAD,), jnp.float32).at[: gamma.shape[0]].set(gamma)
    out = _lookup(t.reshape(_B), gamma_p)
    return out.reshape(_B, 1)


# speedup vs baseline: 4.5811x; 1.0269x over previous
import functools

import jax
import jax.numpy as jnp
from jax import lax
from jax.experimental import pallas as pl
from jax.experimental.pallas import tpu as pltpu
from jax.experimental.pallas import tpu_sc as plsc

_TIMESTEPS = 1000
_B = 16384
_NC = 2
_NS = 16
_LANES = 16
_NW = _NC * _NS
_CHUNK = _B // _NW
_NVEC = _CHUNK // _LANES
_TAB = 1001
_MAGIC = 2.0 ** 23


@functools.partial(
    pl.kernel,
    mesh=plsc.VectorSubcoreMesh(core_axis_name="c", subcore_axis_name="s"),
    out_type=jax.ShapeDtypeStruct((_B,), jnp.float32),
    compiler_params=pltpu.CompilerParams(needs_layout_passes=False),
    scratch_types=[
        pltpu.VMEM((_TAB,), jnp.float32),
        pltpu.VMEM((_CHUNK,), jnp.float32),
        pltpu.VMEM((_CHUNK,), jnp.float32),
        pltpu.SemaphoreType.DMA,
        pltpu.SemaphoreType.DMA,
    ],
)
def _lookup(t_hbm, gamma_hbm, out_hbm, gamma_v, t_v, out_v, sem_g, sem_t):
    wid = lax.axis_index("s") * _NC + lax.axis_index("c")
    base = wid * _CHUNK
    gcp = pltpu.async_copy(gamma_hbm, gamma_v, sem_g)
    tcp = pltpu.async_copy(t_hbm.at[pl.ds(base, _CHUNK)], t_v, sem_t)
    gcp.wait()
    tcp.wait()
    for j in range(_NVEC):
        tv = t_v[pl.ds(j * _LANES, _LANES)]
        x = tv * jnp.float32(_TIMESTEPS)
        r = (x + jnp.float32(_MAGIC)) - jnp.float32(_MAGIC)
        idx = r.astype(jnp.int32)
        idx = jnp.minimum(jnp.maximum(idx, 0), _TIMESTEPS)
        out_v[pl.ds(j * _LANES, _LANES)] = plsc.load_gather(gamma_v, [idx])
    pltpu.sync_copy(out_v, out_hbm.at[pl.ds(base, _CHUNK)])


def kernel(t, gamma):
    out = _lookup(t.reshape(_B), gamma)
    return out.reshape(_B, 1)


# speedup vs baseline: 4.9110x; 1.0720x over previous
import functools

import jax
import jax.numpy as jnp
from jax import lax
from jax.experimental import pallas as pl
from jax.experimental.pallas import tpu as pltpu
from jax.experimental.pallas import tpu_sc as plsc

_TIMESTEPS = 1000
_B = 16384
_NC = 1
_NS = 16
_LANES = 16
_NW = _NC * _NS
_CHUNK = _B // _NW
_NVEC = _CHUNK // _LANES
_TAB = 1001
_MAGIC = 2.0 ** 23


@functools.partial(
    pl.kernel,
    mesh=plsc.VectorSubcoreMesh(
        core_axis_name="c", subcore_axis_name="s", num_cores=_NC
    ),
    out_type=jax.ShapeDtypeStruct((_B,), jnp.float32),
    compiler_params=pltpu.CompilerParams(needs_layout_passes=False),
    scratch_types=[
        pltpu.VMEM((_TAB,), jnp.float32),
        pltpu.VMEM((_CHUNK,), jnp.float32),
        pltpu.VMEM((_CHUNK,), jnp.float32),
        pltpu.SemaphoreType.DMA,
        pltpu.SemaphoreType.DMA,
    ],
)
def _lookup(t_hbm, gamma_hbm, out_hbm, gamma_v, t_v, out_v, sem_g, sem_t):
    wid = lax.axis_index("s") * _NC + lax.axis_index("c")
    base = wid * _CHUNK
    gcp = pltpu.async_copy(gamma_hbm, gamma_v, sem_g)
    tcp = pltpu.async_copy(t_hbm.at[pl.ds(base, _CHUNK)], t_v, sem_t)
    gcp.wait()
    tcp.wait()
    for j in range(_NVEC):
        tv = t_v[pl.ds(j * _LANES, _LANES)]
        x = tv * jnp.float32(_TIMESTEPS)
        r = (x + jnp.float32(_MAGIC)) - jnp.float32(_MAGIC)
        idx = r.astype(jnp.int32)
        idx = jnp.minimum(jnp.maximum(idx, 0), _TIMESTEPS)
        out_v[pl.ds(j * _LANES, _LANES)] = plsc.load_gather(gamma_v, [idx])
    pltpu.sync_copy(out_v, out_hbm.at[pl.ds(base, _CHUNK)])


def kernel(t, gamma):
    out = _lookup(t.reshape(_B), gamma)
    return out.reshape(_B, 1)


# speedup vs baseline: 5.1386x; 1.0463x over previous
import functools

import jax
import jax.numpy as jnp
from jax import lax
from jax.experimental import pallas as pl
from jax.experimental.pallas import tpu as pltpu
from jax.experimental.pallas import tpu_sc as plsc

_TIMESTEPS = 1000
_B = 16384
_NC = 1
_NS = 16
_LANES = 16
_NW = _NC * _NS
_CHUNK = _B // _NW
_NVEC = _CHUNK // _LANES
_TAB = 1001
_MAGIC = 2.0 ** 23


@functools.partial(
    pl.kernel,
    mesh=plsc.VectorSubcoreMesh(
        core_axis_name="c", subcore_axis_name="s", num_cores=_NC
    ),
    out_type=jax.ShapeDtypeStruct((_B,), jnp.float32),
    compiler_params=pltpu.CompilerParams(needs_layout_passes=False),
    scratch_types=[
        pltpu.VMEM((_TAB,), jnp.float32),
        pltpu.VMEM((_CHUNK,), jnp.float32),
        pltpu.VMEM((_CHUNK,), jnp.float32),
        pltpu.SemaphoreType.DMA,
        pltpu.SemaphoreType.DMA,
    ],
)
def _lookup(t_hbm, gamma_hbm, out_hbm, gamma_v, t_v, out_v, sem_g, sem_t):
    wid = lax.axis_index("s") * _NC + lax.axis_index("c")
    base = wid * _CHUNK
    gcp = pltpu.async_copy(gamma_hbm, gamma_v, sem_g)
    tcp = pltpu.async_copy(t_hbm.at[pl.ds(base, _CHUNK)], t_v, sem_t)
    gcp.wait()
    tcp.wait()
    @plsc.parallel_loop(0, _CHUNK, _LANES, unroll=4)
    def _body(off):
        tv = t_v[pl.ds(off, _LANES)]
        x = tv * jnp.float32(_TIMESTEPS)
        r = (x + jnp.float32(_MAGIC)) - jnp.float32(_MAGIC)
        idx = r.astype(jnp.int32)
        idx = jnp.minimum(jnp.maximum(idx, 0), _TIMESTEPS)
        out_v[pl.ds(off, _LANES)] = plsc.load_gather(gamma_v, [idx])
    pltpu.sync_copy(out_v, out_hbm.at[pl.ds(base, _CHUNK)])


def kernel(t, gamma):
    out = _lookup(t.reshape(_B), gamma)
    return out.reshape(_B, 1)


# speedup vs baseline: 5.1795x; 1.0080x over previous
import functools

import jax
import jax.numpy as jnp
from jax import lax
from jax.experimental import pallas as pl
from jax.experimental.pallas import tpu as pltpu
from jax.experimental.pallas import tpu_sc as plsc

_TIMESTEPS = 1000
_B = 16384
_NC = 1
_NS = 16
_LANES = 16
_NW = _NC * _NS
_CHUNK = _B // _NW
_NVEC = _CHUNK // _LANES
_TAB = 1001
_MAGIC = 2.0 ** 23


@functools.partial(
    pl.kernel,
    mesh=plsc.VectorSubcoreMesh(
        core_axis_name="c", subcore_axis_name="s", num_cores=_NC
    ),
    out_type=jax.ShapeDtypeStruct((_B,), jnp.float32),
    compiler_params=pltpu.CompilerParams(needs_layout_passes=False),
    scratch_types=[
        pltpu.VMEM((_TAB,), jnp.float32),
        pltpu.VMEM((_CHUNK,), jnp.float32),
        pltpu.VMEM((_CHUNK,), jnp.float32),
        pltpu.SemaphoreType.DMA,
        pltpu.SemaphoreType.DMA,
    ],
)
def _lookup(t_hbm, gamma_hbm, out_hbm, gamma_v, t_v, out_v, sem_g, sem_t):
    wid = lax.axis_index("s") * _NC + lax.axis_index("c")
    base = wid * _CHUNK
    gcp = pltpu.async_copy(gamma_hbm, gamma_v, sem_g)
    tcp = pltpu.async_copy(t_hbm.at[pl.ds(base, _CHUNK)], t_v, sem_t)
    gcp.wait()
    tcp.wait()
    @plsc.parallel_loop(0, _CHUNK, _LANES, unroll=8)
    def _body(off):
        tv = t_v[pl.ds(off, _LANES)]
        x = tv * jnp.float32(_TIMESTEPS)
        r = (x + jnp.float32(_MAGIC)) - jnp.float32(_MAGIC)
        idx = r.astype(jnp.int32)
        idx = jnp.minimum(jnp.maximum(idx, 0), _TIMESTEPS)
        out_v[pl.ds(off, _LANES)] = plsc.load_gather(gamma_v, [idx])
    pltpu.sync_copy(out_v, out_hbm.at[pl.ds(base, _CHUNK)])


def kernel(t, gamma):
    out = _lookup(t.reshape(_B), gamma)
    return out.reshape(_B, 1)
